# baseline (speedup 1.0000x reference)
import functools

import jax
import jax.numpy as jnp
from jax import lax
from jax.experimental import pallas as pl
from jax.experimental.pallas import tpu as pltpu
from jax.experimental.pallas import tpu_sc as plsc

NC = 2
NS = 16
LN = 16
CH = 128
ROW_BLK = 512

_SC_PARAMS = pltpu.CompilerParams(use_tc_tiling_on_sc=False,
                                  needs_layout_passes=False)


def _sc_mesh():
  return plsc.VectorSubcoreMesh(core_axis_name="c", subcore_axis_name="s")


def _make_sc_layer(n_pad, e_pad, f):
  n_tile = n_pad // NS
  epw = e_pad // (NC * NS)
  nch = epw // CH
  nj = f // LN

  @functools.partial(
      pl.kernel,
      out_type=jax.ShapeDtypeStruct((NC, n_pad, f), jnp.float32),
      mesh=_sc_mesh(),
      compiler_params=_SC_PARAMS,
      scratch_types=[
          pltpu.MemorySpace.VMEM_SHARED((n_pad, f), jnp.float32),
          pltpu.VMEM((2, CH, f), jnp.float32),
          pltpu.VMEM((2, 3, CH), jnp.int32),
          pltpu.VMEM((2, CH), jnp.int32),
          pltpu.VMEM((2, CH), jnp.int32),
          pltpu.SemaphoreType.DMA,
          pltpu.SemaphoreType.DMA,
      ],
  )
  def sc_layer(hrel_hbm, esdt_hbm, out_hbm, acc, rows2, idxs, cidx, dsts,
               sem_g, sem_s):
    c = lax.axis_index("c")
    s = lax.axis_index("s")
    w = c * NS + s

    z = jnp.zeros((LN,), jnp.float32)

    def zrow(i, carry):
      for j in range(nj):
        rows2[0, i, pl.ds(j * LN, LN)] = z
      return carry

    lax.fori_loop(0, CH, zrow, 0)
    for t in range(n_tile // CH):
      pltpu.sync_copy(rows2.at[0], acc.at[pl.ds(s * n_tile + t * CH, CH)])

    plsc.subcore_barrier()

    base_w = w * nch

    def load_and_combine(g, q):
      pltpu.sync_copy(esdt_hbm.at[base_w + g], idxs.at[q])
      for k in range(CH // LN):
        sl = pl.ds(k * LN, LN)
        cidx[q, sl] = idxs[q, 2, sl] * n_pad + idxs[q, 0, sl]
        dsts[q, sl] = idxs[q, 1, sl]

    load_and_combine(0, 0)
    pltpu.async_copy(hrel_hbm.at[cidx.at[0]], rows2.at[0], sem_g)

    def chunk(g, carry):
      p = lax.rem(g, 2)
      q = 1 - p

      @pl.when(g + 1 < nch)
      def _():
        @pl.when(g > 0)
        def _():
          pltpu.make_async_copy(rows2.at[q], acc.at[dsts.at[q]],
                                sem_s).wait()

        load_and_combine(g + 1, q)
        pltpu.async_copy(hrel_hbm.at[cidx.at[q]], rows2.at[q], sem_g)

      pltpu.make_async_copy(hrel_hbm.at[cidx.at[p]], rows2.at[p],
                            sem_g).wait()
      pltpu.async_copy(rows2.at[p], acc.at[dsts.at[p]], sem_s, add=True)
      return carry

    lax.fori_loop(0, nch, chunk, 0)
    pltpu.make_async_copy(rows2.at[0], acc.at[dsts.at[0]], sem_s).wait()
    pltpu.make_async_copy(rows2.at[1], acc.at[dsts.at[1]], sem_s).wait()
    plsc.subcore_barrier()

    for t in range(n_tile // CH):
      off = s * n_tile + t * CH
      pltpu.sync_copy(acc.at[pl.ds(off, CH)], out_hbm.at[c, pl.ds(off, CH)])

  return sc_layer


def _make_sc_count(n_pad, e_pad):
  n_tile = n_pad // NS
  epw = e_pad // (NC * NS)
  nch = epw // CH

  @functools.partial(
      pl.kernel,
      out_type=jax.ShapeDtypeStruct((NC, n_pad, LN), jnp.float32),
      mesh=_sc_mesh(),
      compiler_params=_SC_PARAMS,
      scratch_types=[
          pltpu.MemorySpace.VMEM_SHARED((n_pad, LN), jnp.float32),
          pltpu.VMEM((CH, LN), jnp.float32),
          pltpu.VMEM((CH,), jnp.int32),
      ],
  )
  def sc_count(dst_hbm, out_hbm, acc, ones_v, dst_v):
    c = lax.axis_index("c")
    s = lax.axis_index("s")
    w = c * NS + s

    z = jnp.zeros((LN,), jnp.float32)

    def zrow(i, carry):
      ones_v[i, :] = z
      return carry

    lax.fori_loop(0, CH, zrow, 0)
    for t in range(n_tile // CH):
      pltpu.sync_copy(ones_v, acc.at[pl.ds(s * n_tile + t * CH, CH)])

    one = jnp.ones((LN,), jnp.float32)

    def orow(i, carry):
      ones_v[i, :] = one
      return carry

    lax.fori_loop(0, CH, orow, 0)
    plsc.subcore_barrier()

    base_w = w * epw

    def chunk(g, carry):
      b = base_w + g * CH
      pltpu.sync_copy(dst_hbm.at[pl.ds(b, CH)], dst_v)
      pltpu.sync_copy(ones_v, acc.at[dst_v], add=True)
      return carry

    lax.fori_loop(0, nch, chunk, 0)
    plsc.subcore_barrier()

    for t in range(n_tile // CH):
      off = s * n_tile + t * CH
      pltpu.sync_copy(acc.at[pl.ds(off, CH)], out_hbm.at[c, pl.ds(off, CH)])

  return sc_count


def _type_embed(nt_oh, type_emb):
  n_pad, nt = nt_oh.shape
  ted = type_emb.shape[1]
  grid = n_pad // ROW_BLK

  def body(oh_ref, te_ref, out_ref):
    out_ref[...] = jnp.dot(oh_ref[...], te_ref[...],
                           preferred_element_type=jnp.float32)

  return pl.pallas_call(
      body,
      grid=(grid,),
      in_specs=[
          pl.BlockSpec((ROW_BLK, nt), lambda i: (i, 0)),
          pl.BlockSpec((nt, ted), lambda i: (0, 0)),
      ],
      out_specs=pl.BlockSpec((ROW_BLK, ted), lambda i: (i, 0)),
      out_shape=jax.ShapeDtypeStruct((n_pad, ted), jnp.float32),
  )(nt_oh, type_emb)


def _expand_rel(h, relp):
  n_pad, f = h.shape
  grid = n_pad // ROW_BLK

  def body(h_ref, rel_ref, out_ref):
    hv = h_ref[...]
    for r in range(16):
      out_ref[r] = hv * rel_ref[r:r + 1, :]

  return pl.pallas_call(
      body,
      grid=(grid,),
      in_specs=[
          pl.BlockSpec((ROW_BLK, f), lambda i: (i, 0)),
          pl.BlockSpec((16, f), lambda i: (0, 0)),
      ],
      out_specs=pl.BlockSpec((16, ROW_BLK, f), lambda i: (0, i, 0)),
      out_shape=jax.ShapeDtypeStruct((16, n_pad, f), jnp.float32),
  )(h, relp)


def _combine(parts, cnt2, relu):
  n_pad = parts[0][1].shape[0]
  o = parts[0][3].shape[1]
  grid = n_pad // ROW_BLK
  np_ = len(parts)

  def body(*refs):
    out_ref = refs[-1]
    cnt_ref = refs[5 * np_]
    cnt = cnt_ref[0, :, 0:1] + cnt_ref[1, :, 0:1] + 1.0
    inv = 1.0 / jnp.maximum(cnt, 1.0)
    out = None
    for p in range(np_):
      acc_ref, h_ref, rel_ref, wo_ref, wl_ref = refs[5 * p: 5 * p + 5]
      hv = h_ref[...]
      summed = acc_ref[0] + acc_ref[1] + hv * rel_ref[1:2, :]
      agg = summed * inv
      t = (jnp.dot(agg, wo_ref[...], preferred_element_type=jnp.float32)
           + jnp.dot(hv, wl_ref[...], preferred_element_type=jnp.float32))
      out = t if out is None else out + t
    if relu:
      out = jnp.maximum(out, 0.0)
    out_ref[...] = out

  in_specs = []
  args = []
  for (acc2, h, rel, w_out, w_loop) in parts:
    fp = h.shape[1]
    in_specs += [
        pl.BlockSpec((NC, ROW_BLK, fp), lambda i: (0, i, 0)),
        pl.BlockSpec((ROW_BLK, fp), lambda i: (i, 0)),
        pl.BlockSpec((16, fp), lambda i: (0, 0)),
        pl.BlockSpec((fp, o), lambda i: (0, 0)),
        pl.BlockSpec((fp, o), lambda i: (0, 0)),
    ]
    args += [acc2, h, rel, w_out, w_loop]
  in_specs.append(pl.BlockSpec((NC, ROW_BLK, LN), lambda i: (0, i, 0)))
  args.append(cnt2)

  return pl.pallas_call(
      body,
      grid=(grid,),
      in_specs=in_specs,
      out_specs=pl.BlockSpec((ROW_BLK, o), lambda i: (i, 0)),
      out_shape=jax.ShapeDtypeStruct((n_pad, o), jnp.float32),
  )(*args)


def kernel(x, node_type_ids, edge_index, edge_type, type_emb,
           rel1, w_loop1, w_out1, rel2, w_loop2, w_out2,
           rel3, w_loop3, w_out3):
  n, d = x.shape
  e = edge_type.shape[0]
  nt = type_emb.shape[0]
  ted = type_emb.shape[1]

  n_pad = ((n + ROW_BLK - 1) // ROW_BLK) * ROW_BLK
  chunk_all = NC * NS * CH
  e_pad = ((e + chunk_all - 1) // chunk_all) * chunk_all

  x_pad = jnp.zeros((n_pad, d), jnp.float32).at[:n].set(x)
  nt_oh = jnp.zeros((n_pad, nt), jnp.float32).at[:n].set(
      (node_type_ids[:, None] == jnp.arange(nt)[None, :]).astype(jnp.float32))

  npadrows = max(n_pad - n, 1)
  pad_cnt = e_pad - e
  pad_idx = (n + (jnp.arange(pad_cnt, dtype=jnp.int32) % npadrows)
             ).astype(jnp.int32)
  src = jnp.concatenate([edge_index[0].astype(jnp.int32), pad_idx])
  dst = jnp.concatenate([edge_index[1].astype(jnp.int32), pad_idx])
  et = jnp.concatenate([edge_type.astype(jnp.int32),
                        jnp.zeros((pad_cnt,), jnp.int32)])
  esdt = jnp.stack([src, dst, et]).reshape(3, e_pad // CH, CH).transpose(1, 0, 2)

  def pad16(r):
    return jnp.zeros((16, r.shape[1]), jnp.float32).at[:r.shape[0]].set(r)

  rel1p, rel2p, rel3p = pad16(rel1), pad16(rel2), pad16(rel3)
  rel1a, rel1b = rel1p[:, :d], rel1p[:, d:]
  w_out1a, w_out1b = w_out1[:d], w_out1[d:]
  w_loop1a, w_loop1b = w_loop1[:d], w_loop1[d:]

  te = _type_embed(nt_oh, type_emb)
  cnt2 = _make_sc_count(n_pad, e_pad)(dst)

  sc_d = _make_sc_layer(n_pad, e_pad, d)
  sc_t = _make_sc_layer(n_pad, e_pad, ted)

  def expand(h, relp):
    return _expand_rel(h, relp).reshape(16 * n_pad, h.shape[1])

  acc1a = sc_d(expand(x_pad, rel1a), esdt)
  acc1b = sc_t(expand(te, rel1b), esdt)
  h1 = _combine([(acc1a, x_pad, rel1a, w_out1a, w_loop1a),
                 (acc1b, te, rel1b, w_out1b, w_loop1b)], cnt2, relu=True)

  acc2 = sc_d(expand(h1, rel2p), esdt)
  h2 = _combine([(acc2, h1, rel2p, w_out2, w_loop2)], cnt2, relu=True)

  acc3 = sc_d(expand(h2, rel3p), esdt)
  h3 = _combine([(acc3, h2, rel3p, w_out3, w_loop3)], cnt2, relu=False)

  return h3[:n]


# speedup vs baseline: 1.0969x; 1.0969x over previous
import functools

import jax
import jax.numpy as jnp
from jax import lax
from jax.experimental import pallas as pl
from jax.experimental.pallas import tpu as pltpu
from jax.experimental.pallas import tpu_sc as plsc

NC = 2
NS = 16
LN = 16
CH = 128
ROW_BLK = 512

_SC_PARAMS = pltpu.CompilerParams(use_tc_tiling_on_sc=False,
                                  needs_layout_passes=False)


def _sc_mesh():
  return plsc.VectorSubcoreMesh(core_axis_name="c", subcore_axis_name="s")


def _make_sc_layer(n_pad, e_pad, f, with_count=False):
  n_tile = n_pad // NS
  epw = e_pad // (NC * NS)
  nch = epw // CH
  nj = f // LN

  out_types = [jax.ShapeDtypeStruct((NC, n_pad, f), jnp.float32)]
  scratch = [
      pltpu.MemorySpace.VMEM_SHARED((n_pad, f), jnp.float32),
      pltpu.VMEM((2, CH, f), jnp.float32),
      pltpu.VMEM((2, 3, CH), jnp.int32),
      pltpu.VMEM((2, CH), jnp.int32),
      pltpu.VMEM((2, CH), jnp.int32),
      pltpu.SemaphoreType.DMA,
      pltpu.SemaphoreType.DMA,
  ]
  if with_count:
    out_types.append(jax.ShapeDtypeStruct((NC, n_pad, LN), jnp.float32))
    scratch.append(pltpu.MemorySpace.VMEM_SHARED((n_pad, LN), jnp.float32))
    scratch.append(pltpu.VMEM((CH, LN), jnp.float32))

  @functools.partial(
      pl.kernel,
      out_type=tuple(out_types) if with_count else out_types[0],
      mesh=_sc_mesh(),
      compiler_params=_SC_PARAMS,
      scratch_types=scratch,
  )
  def sc_layer(hrel_hbm, esdt_hbm, *refs):
    if with_count:
      (out_hbm, cnt_hbm, acc, rows2, idxs, cidx, dsts, sem_g, sem_s,
       acc_cnt, ones_v) = refs
    else:
      out_hbm, acc, rows2, idxs, cidx, dsts, sem_g, sem_s = refs
    c = lax.axis_index("c")
    s = lax.axis_index("s")
    w = c * NS + s

    z = jnp.zeros((LN,), jnp.float32)

    def zrow(i, carry):
      for j in range(nj):
        rows2[0, i, pl.ds(j * LN, LN)] = z
      return carry

    lax.fori_loop(0, CH, zrow, 0)
    for t in range(n_tile // CH):
      pltpu.sync_copy(rows2.at[0], acc.at[pl.ds(s * n_tile + t * CH, CH)])

    if with_count:
      one = jnp.ones((LN,), jnp.float32)

      def orow(i, carry):
        ones_v[i, :] = z
        return carry

      lax.fori_loop(0, CH, orow, 0)
      for t in range(n_tile // CH):
        pltpu.sync_copy(ones_v, acc_cnt.at[pl.ds(s * n_tile + t * CH, CH)])

      def orow2(i, carry):
        ones_v[i, :] = one
        return carry

      lax.fori_loop(0, CH, orow2, 0)

    plsc.subcore_barrier()

    base_w = w * nch

    def load_and_combine(g, q):
      pltpu.sync_copy(esdt_hbm.at[base_w + g], idxs.at[q])
      for k in range(CH // LN):
        sl = pl.ds(k * LN, LN)
        cidx[q, sl] = idxs[q, 2, sl] * n_pad + idxs[q, 0, sl]
        dsts[q, sl] = idxs[q, 1, sl]

    load_and_combine(0, 0)
    pltpu.async_copy(hrel_hbm.at[cidx.at[0]], rows2.at[0], sem_g)

    def chunk(g, carry):
      p = lax.rem(g, 2)
      q = 1 - p

      @pl.when(g + 1 < nch)
      def _():
        @pl.when(g > 0)
        def _():
          pltpu.make_async_copy(rows2.at[q], acc.at[dsts.at[q]],
                                sem_s).wait()

        load_and_combine(g + 1, q)
        pltpu.async_copy(hrel_hbm.at[cidx.at[q]], rows2.at[q], sem_g)

      pltpu.make_async_copy(hrel_hbm.at[cidx.at[p]], rows2.at[p],
                            sem_g).wait()
      pltpu.async_copy(rows2.at[p], acc.at[dsts.at[p]], sem_s, add=True)
      if with_count:
        pltpu.sync_copy(ones_v, acc_cnt.at[dsts.at[p]], add=True)
      return carry

    lax.fori_loop(0, nch, chunk, 0)
    pltpu.make_async_copy(rows2.at[0], acc.at[dsts.at[0]], sem_s).wait()
    pltpu.make_async_copy(rows2.at[1], acc.at[dsts.at[1]], sem_s).wait()
    plsc.subcore_barrier()

    for t in range(n_tile // CH):
      off = s * n_tile + t * CH
      pltpu.sync_copy(acc.at[pl.ds(off, CH)], out_hbm.at[c, pl.ds(off, CH)])
      if with_count:
        pltpu.sync_copy(acc_cnt.at[pl.ds(off, CH)],
                        cnt_hbm.at[c, pl.ds(off, CH)])

  return sc_layer


def _type_embed(nt_oh, type_emb):
  n_pad, nt = nt_oh.shape
  ted = type_emb.shape[1]
  grid = n_pad // ROW_BLK

  def body(oh_ref, te_ref, out_ref):
    out_ref[...] = jnp.dot(oh_ref[...], te_ref[...],
                           preferred_element_type=jnp.float32)

  return pl.pallas_call(
      body,
      grid=(grid,),
      in_specs=[
          pl.BlockSpec((ROW_BLK, nt), lambda i: (i, 0)),
          pl.BlockSpec((nt, ted), lambda i: (0, 0)),
      ],
      out_specs=pl.BlockSpec((ROW_BLK, ted), lambda i: (i, 0)),
      out_shape=jax.ShapeDtypeStruct((n_pad, ted), jnp.float32),
  )(nt_oh, type_emb)


def _expand_rel(h, relp):
  n_pad, f = h.shape
  grid = n_pad // ROW_BLK

  def body(h_ref, rel_ref, out_ref):
    hv = h_ref[...]
    for r in range(16):
      out_ref[r] = hv * rel_ref[r:r + 1, :]

  return pl.pallas_call(
      body,
      grid=(grid,),
      in_specs=[
          pl.BlockSpec((ROW_BLK, f), lambda i: (i, 0)),
          pl.BlockSpec((16, f), lambda i: (0, 0)),
      ],
      out_specs=pl.BlockSpec((16, ROW_BLK, f), lambda i: (0, i, 0)),
      out_shape=jax.ShapeDtypeStruct((16, n_pad, f), jnp.float32),
  )(h, relp)


def _combine(parts, cnt2, relu, rel_next=None):
  n_pad = parts[0][1].shape[0]
  o = parts[0][3].shape[1]
  grid = n_pad // ROW_BLK
  np_ = len(parts)

  def body(*refs):
    cnt_ref = refs[5 * np_]
    cnt = cnt_ref[0, :, 0:1] + cnt_ref[1, :, 0:1] + 1.0
    inv = 1.0 / jnp.maximum(cnt, 1.0)
    out = None
    for p in range(np_):
      acc_ref, h_ref, rel_ref, wo_ref, wl_ref = refs[5 * p: 5 * p + 5]
      hv = h_ref[...]
      summed = acc_ref[0] + acc_ref[1] + hv * rel_ref[1:2, :]
      agg = summed * inv
      t = (jnp.dot(agg, wo_ref[...], preferred_element_type=jnp.float32)
           + jnp.dot(hv, wl_ref[...], preferred_element_type=jnp.float32))
      out = t if out is None else out + t
    if relu:
      out = jnp.maximum(out, 0.0)
    if rel_next is None:
      refs[-1][...] = out
    else:
      out_ref, hrel_ref = refs[-2], refs[-1]
      relnext_ref = refs[5 * np_ + 1]
      out_ref[...] = out
      for r in range(16):
        hrel_ref[r] = out * relnext_ref[r:r + 1, :]

  in_specs = []
  args = []
  for (acc2, h, rel, w_out, w_loop) in parts:
    fp = h.shape[1]
    in_specs += [
        pl.BlockSpec((NC, ROW_BLK, fp), lambda i: (0, i, 0)),
        pl.BlockSpec((ROW_BLK, fp), lambda i: (i, 0)),
        pl.BlockSpec((16, fp), lambda i: (0, 0)),
        pl.BlockSpec((fp, o), lambda i: (0, 0)),
        pl.BlockSpec((fp, o), lambda i: (0, 0)),
    ]
    args += [acc2, h, rel, w_out, w_loop]
  in_specs.append(pl.BlockSpec((NC, ROW_BLK, LN), lambda i: (0, i, 0)))
  args.append(cnt2)

  out_specs = pl.BlockSpec((ROW_BLK, o), lambda i: (i, 0))
  out_shape = jax.ShapeDtypeStruct((n_pad, o), jnp.float32)
  if rel_next is not None:
    in_specs.append(pl.BlockSpec((16, o), lambda i: (0, 0)))
    args.append(rel_next)
    out_specs = [out_specs, pl.BlockSpec((16, ROW_BLK, o), lambda i: (0, i, 0))]
    out_shape = [out_shape, jax.ShapeDtypeStruct((16, n_pad, o), jnp.float32)]

  return pl.pallas_call(
      body,
      grid=(grid,),
      in_specs=in_specs,
      out_specs=out_specs,
      out_shape=out_shape,
  )(*args)


def kernel(x, node_type_ids, edge_index, edge_type, type_emb,
           rel1, w_loop1, w_out1, rel2, w_loop2, w_out2,
           rel3, w_loop3, w_out3):
  n, d = x.shape
  e = edge_type.shape[0]
  nt = type_emb.shape[0]
  ted = type_emb.shape[1]

  n_pad = ((n + ROW_BLK - 1) // ROW_BLK) * ROW_BLK
  chunk_all = NC * NS * CH
  e_pad = ((e + chunk_all - 1) // chunk_all) * chunk_all

  x_pad = jnp.zeros((n_pad, d), jnp.float32).at[:n].set(x)
  nt_oh = jnp.zeros((n_pad, nt), jnp.float32).at[:n].set(
      (node_type_ids[:, None] == jnp.arange(nt)[None, :]).astype(jnp.float32))

  npadrows = max(n_pad - n, 1)
  pad_cnt = e_pad - e
  pad_idx = (n + (jnp.arange(pad_cnt, dtype=jnp.int32) % npadrows)
             ).astype(jnp.int32)
  src = jnp.concatenate([edge_index[0].astype(jnp.int32), pad_idx])
  dst = jnp.concatenate([edge_index[1].astype(jnp.int32), pad_idx])
  et = jnp.concatenate([edge_type.astype(jnp.int32),
                        jnp.zeros((pad_cnt,), jnp.int32)])
  esdt = jnp.stack([src, dst, et]).reshape(3, e_pad // CH, CH).transpose(1, 0, 2)

  def pad16(r):
    return jnp.zeros((16, r.shape[1]), jnp.float32).at[:r.shape[0]].set(r)

  rel1p, rel2p, rel3p = pad16(rel1), pad16(rel2), pad16(rel3)
  rel1a, rel1b = rel1p[:, :d], rel1p[:, d:]
  w_out1a, w_out1b = w_out1[:d], w_out1[d:]
  w_loop1a, w_loop1b = w_loop1[:d], w_loop1[d:]

  te = _type_embed(nt_oh, type_emb)

  sc_d = _make_sc_layer(n_pad, e_pad, d)
  sc_t = _make_sc_layer(n_pad, e_pad, ted, with_count=True)

  def expand(h, relp):
    return _expand_rel(h, relp).reshape(16 * n_pad, h.shape[1])

  acc1a = sc_d(expand(x_pad, rel1a), esdt)
  acc1b, cnt2 = sc_t(expand(te, rel1b), esdt)
  h1, hrel2 = _combine([(acc1a, x_pad, rel1a, w_out1a, w_loop1a),
                        (acc1b, te, rel1b, w_out1b, w_loop1b)],
                       cnt2, relu=True, rel_next=rel2p)

  acc2 = sc_d(hrel2.reshape(16 * n_pad, d), esdt)
  h2, hrel3 = _combine([(acc2, h1, rel2p, w_out2, w_loop2)],
                       cnt2, relu=True, rel_next=rel3p)

  acc3 = sc_d(hrel3.reshape(16 * n_pad, d), esdt)
  h3 = _combine([(acc3, h2, rel3p, w_out3, w_loop3)], cnt2, relu=False)

  return h3[:n]
